# baseline (speedup 1.0000x reference)
import functools

import jax
import jax.numpy as jnp
from jax import lax
from jax.experimental import pallas as pl
from jax.experimental.pallas import tpu as pltpu
from jax.experimental.pallas import tpu_sc as plsc

VOCAB = 1000000
EMBED = 64
HIDDEN = 128
OUT = 2
B = 4096
L = 200

NC = 2
NS = 16
NW = NC * NS
ROWS_PER_TILE = B // NW


def _sc_pool(x, emb):
    mesh = plsc.VectorSubcoreMesh(
        core_axis_name="c", subcore_axis_name="s", num_cores=NC, num_subcores=NS
    )

    @functools.partial(
        pl.kernel,
        out_type=jax.ShapeDtypeStruct((B, EMBED), jnp.float32),
        mesh=mesh,
        compiler_params=pltpu.CompilerParams(use_tc_tiling_on_sc=False),
        scratch_types=[
            pltpu.VMEM((ROWS_PER_TILE, L), jnp.int32),
            pltpu.VMEM((2, L, EMBED), jnp.float32),
            pltpu.VMEM((ROWS_PER_TILE, EMBED), jnp.float32),
            pltpu.SemaphoreType.DMA,
            pltpu.SemaphoreType.DMA,
            pltpu.SemaphoreType.DMA,
        ],
    )
    def pool_kernel(
        x_hbm, emb_hbm, out_hbm, idx_v, rows_v, pooled_v, sem_i, sem_g0, sem_g1
    ):
        wid = lax.axis_index("s") * NC + lax.axis_index("c")
        row_base = wid * ROWS_PER_TILE
        pltpu.async_copy(
            x_hbm.at[pl.ds(row_base, ROWS_PER_TILE)], idx_v, sem_i
        ).wait()
        sems = (sem_g0, sem_g1)

        def gather_descs(r, b):
            d1 = pltpu.make_async_copy(
                emb_hbm.at[idx_v.at[r, pl.ds(0, 128)]],
                rows_v.at[b].at[pl.ds(0, 128)],
                sems[b],
            )
            d2 = pltpu.make_async_copy(
                emb_hbm.at[idx_v.at[r, pl.ds(128, L - 128)]],
                rows_v.at[b].at[pl.ds(128, L - 128)],
                sems[b],
            )
            return d1, d2

        def issue(r, b):
            d1, d2 = gather_descs(r, b)
            d1.start()
            d2.start()

        issue(0, 0)
        issue(1, 1)

        @pl.loop(0, ROWS_PER_TILE // 2)
        def _pair(t):
            for b in range(2):
                r = 2 * t + b
                d1, d2 = gather_descs(r, b)
                d1.wait()
                d2.wait()

                @pl.when(r + 2 < ROWS_PER_TILE)
                def _():
                    issue(r + 2, b)

                buf = rows_v.at[b]
                zeros = jnp.zeros((16,), jnp.float32)

                def body(k8, accs):
                    accs = list(accs)
                    for j in range(8):
                        k = k8 * 8 + j
                        for c in range(4):
                            i = c * 2 + (j & 1)
                            accs[i] = accs[i] + buf[k, pl.ds(c * 16, 16)]
                    return tuple(accs)

                accs = lax.fori_loop(0, L // 8, body, (zeros,) * 8)
                for c in range(4):
                    pooled_v[r, pl.ds(c * 16, 16)] = accs[c * 2] + accs[c * 2 + 1]

        pltpu.sync_copy(
            pooled_v, out_hbm.at[pl.ds(row_base, ROWS_PER_TILE)]
        )

    return pool_kernel(x, emb)


def _tc_mlp(pooled, W1t, b1, W2t, b2):
    def mlp_body(p_ref, w1_ref, b1_ref, w2_ref, b2_ref, o_ref):
        p = p_ref[...] * (1.0 / L)
        h = jnp.dot(p, w1_ref[...], preferred_element_type=jnp.float32)
        h = jnp.maximum(h + b1_ref[...], 0.0)
        o_ref[...] = (
            jnp.dot(h, w2_ref[...], preferred_element_type=jnp.float32)
            + b2_ref[...]
        )

    return pl.pallas_call(
        mlp_body,
        out_shape=jax.ShapeDtypeStruct((B, OUT), jnp.float32),
    )(pooled, W1t, b1, W2t, b2)


def kernel(x, emb, W1, b1, W2, b2):
    pooled = _sc_pool(x, emb)
    return _tc_mlp(
        pooled,
        W1.T,
        b1.reshape(1, HIDDEN),
        W2.T,
        b2.reshape(1, OUT),
    )


# speedup vs baseline: 1.0005x; 1.0005x over previous
import functools

import jax
import jax.numpy as jnp
from jax import lax
from jax.experimental import pallas as pl
from jax.experimental.pallas import tpu as pltpu
from jax.experimental.pallas import tpu_sc as plsc

VOCAB = 1000000
EMBED = 64
HIDDEN = 128
OUT = 2
B = 4096
L = 200

NC = 2
NS = 16
NW = NC * NS
ROWS_PER_TILE = B // NW
LPAD = 256


def _sc_pool(x, emb):
    mesh = plsc.VectorSubcoreMesh(
        core_axis_name="c", subcore_axis_name="s", num_cores=NC, num_subcores=NS
    )

    @functools.partial(
        pl.kernel,
        out_type=jax.ShapeDtypeStruct((B, EMBED), jnp.float32),
        mesh=mesh,
        compiler_params=pltpu.CompilerParams(use_tc_tiling_on_sc=False),
        scratch_types=[
            pltpu.VMEM((ROWS_PER_TILE, LPAD), jnp.int32),
            pltpu.VMEM((2, L, EMBED), jnp.float32),
            pltpu.VMEM((ROWS_PER_TILE, EMBED), jnp.float32),
            pltpu.SemaphoreType.DMA,
            pltpu.SemaphoreType.DMA,
            pltpu.SemaphoreType.DMA,
        ],
    )
    def pool_kernel(
        x_hbm, emb_hbm, out_hbm, idx_v, rows_v, pooled_v, sem_i, sem_g0, sem_g1
    ):
        wid = lax.axis_index("s") * NC + lax.axis_index("c")
        row_base = wid * ROWS_PER_TILE
        pltpu.async_copy(
            x_hbm.at[pl.ds(row_base, ROWS_PER_TILE)], idx_v, sem_i
        ).wait()
        sems = (sem_g0, sem_g1)

        def gather_descs(r, b):
            d1 = pltpu.make_async_copy(
                emb_hbm.at[idx_v.at[r, pl.ds(0, 128)]],
                rows_v.at[b].at[pl.ds(0, 128)],
                sems[b],
            )
            d2 = pltpu.make_async_copy(
                emb_hbm.at[idx_v.at[r, pl.ds(128, L - 128)]],
                rows_v.at[b].at[pl.ds(128, L - 128)],
                sems[b],
            )
            return d1, d2

        def issue(r, b):
            d1, d2 = gather_descs(r, b)
            d1.start()
            d2.start()

        issue(0, 0)
        issue(1, 1)

        @pl.loop(0, ROWS_PER_TILE // 2)
        def _pair(t):
            for b in range(2):
                r = 2 * t + b
                d1, d2 = gather_descs(r, b)
                d1.wait()
                d2.wait()

                @pl.when(r + 2 < ROWS_PER_TILE)
                def _():
                    issue(r + 2, b)

                buf = rows_v.at[b]
                zeros = jnp.zeros((16,), jnp.float32)

                def body(k8, accs):
                    accs = list(accs)
                    for j in range(8):
                        k = k8 * 8 + j
                        for c in range(4):
                            i = c * 2 + (j & 1)
                            accs[i] = accs[i] + buf[k, pl.ds(c * 16, 16)]
                    return tuple(accs)

                accs = lax.fori_loop(0, L // 8, body, (zeros,) * 8)
                for c in range(4):
                    pooled_v[r, pl.ds(c * 16, 16)] = accs[c * 2] + accs[c * 2 + 1]

        pltpu.sync_copy(
            pooled_v, out_hbm.at[pl.ds(row_base, ROWS_PER_TILE)]
        )

    return pool_kernel(x, emb)


def _tc_mlp(pooled, W1t, b1, W2t, b2):
    def mlp_body(p_ref, w1_ref, b1_ref, w2_ref, b2_ref, o_ref):
        p = p_ref[...] * (1.0 / L)
        h = jnp.dot(p, w1_ref[...], preferred_element_type=jnp.float32)
        h = jnp.maximum(h + b1_ref[...], 0.0)
        o_ref[...] = (
            jnp.dot(h, w2_ref[...], preferred_element_type=jnp.float32)
            + b2_ref[...]
        )

    return pl.pallas_call(
        mlp_body,
        out_shape=jax.ShapeDtypeStruct((B, OUT), jnp.float32),
    )(pooled, W1t, b1, W2t, b2)


def kernel(x, emb, W1, b1, W2, b2):
    xp = jnp.pad(x, ((0, 0), (0, LPAD - L)))
    pooled = _sc_pool(xp, emb)
    return _tc_mlp(
        pooled,
        W1.T,
        b1.reshape(1, HIDDEN),
        W2.T,
        b2.reshape(1, OUT),
    )
